# baseline (speedup 1.0000x reference)
import functools

import jax
import jax.numpy as jnp
from jax import lax
from jax.experimental import pallas as pl
from jax.experimental.pallas import tpu as pltpu
from jax.experimental.pallas import tpu_sc as plsc

EMBED_DIM = 64
BATCH = 4096
T = 30
NF = 6
OUT_D = (NF - 1) * EMBED_DIM

NC = 2
NS = 16
NW = NC * NS
PAIRS_PER_W = BATCH // (2 * NW)
IDX_PER_PAIR = 2 * T * NF


def _rsqrt16(s):
    i = lax.bitcast_convert_type(s, jnp.int32)
    y = lax.bitcast_convert_type(jnp.int32(0x5F3759DF) - (i >> 1), jnp.float32)
    for _ in range(3):
        y = y * (jnp.float32(1.5) - jnp.float32(0.5) * s * y * y)
    return y


def _body(x_hbm, table_hbm, out_hbm,
          i0, i1, i2, r0_, r1_, r2_, o0, o1, o2,
          si0, si1, si2, sg0, sg1, sg2, so0, so1, so2):
    wid = lax.axis_index("s") * NC + lax.axis_index("c")
    base_p = wid * PAIRS_PER_W
    ibuf = (i0, i1, i2)
    rbuf = (r0_, r1_, r2_)
    obuf = (o0, o1, o2)
    sem_i = (si0, si1, si2)
    sem_g = (sg0, sg1, sg2)
    sem_o = (so0, so1, so2)

    def fire_idx(p, j):
        pltpu.async_copy(x_hbm.at[pl.ds(base_p + p, 1)], ibuf[j], sem_i[j])

    def wait_idx(j):
        pltpu.make_async_copy(
            x_hbm.at[pl.ds(0, 1)], ibuf[j], sem_i[j]).wait()

    def fire_gather(j):
        pltpu.async_copy(table_hbm.at[ibuf[j].at[0]], rbuf[j], sem_g[j])

    def wait_gather(j):
        pltpu.make_async_copy(
            table_hbm.at[ibuf[j].at[0]], rbuf[j], sem_g[j]).wait()

    def fire_out(p, j):
        pltpu.async_copy(
            obuf[j], out_hbm.at[pl.ds((base_p + p) * 2, 2)], sem_o[j])

    def wait_out(j):
        pltpu.make_async_copy(
            obuf[j], out_hbm.at[pl.ds(0, 2)], sem_o[j]).wait()

    def compute(j):
        rows_v = rbuf[j]
        out_v = obuf[j]
        for be in range(2):
            def t_body(t, acc):
                base = (be * T + t) * NF
                f5 = [rows_v[base + 5, pl.ds(jj * 16, 16)] for jj in range(4)]
                new_acc = list(acc)
                for i in range(5):
                    for jj in range(4):
                        v = rows_v[base + i, pl.ds(jj * 16, 16)] + f5[jj]
                        out_v[be, t, pl.ds(i * 64 + jj * 16, 16)] = v
                        k = i * 4 + jj
                        new_acc[k] = acc[k] + v * v
                return tuple(new_acc)

            zero = jnp.zeros((16,), jnp.float32)
            acc = lax.fori_loop(0, T, t_body, tuple(zero for _ in range(20)))

            scales = []
            for k in range(20):
                s = acc[k]
                y = _rsqrt16(s)
                scales.append(
                    jnp.where(s >= jnp.float32(1e-24), y, jnp.float32(1e12))
                )

            def scale_body(t, carry2):
                for i in range(5):
                    for jj in range(4):
                        sl = pl.ds(i * 64 + jj * 16, 16)
                        out_v[be, t, sl] = out_v[be, t, sl] * scales[i * 4 + jj]
                return carry2

            lax.fori_loop(0, T, scale_body, 0)

    pltpu.sync_copy(x_hbm.at[pl.ds(base_p + 0, 1)], ibuf[0])
    pltpu.sync_copy(x_hbm.at[pl.ds(base_p + 1, 1)], ibuf[1])
    fire_gather(0)
    fire_gather(1)
    fire_idx(2, 2)

    def step_body(s, carry):
        for j in range(3):
            p = s * 3 + j

            @pl.when(p + 3 < PAIRS_PER_W)
            def _():
                fire_idx(p + 3, j)

            @pl.when(p + 2 < PAIRS_PER_W)
            def _():
                wait_idx((j + 2) % 3)
                fire_gather((j + 2) % 3)

            @pl.when(p < PAIRS_PER_W)
            def _():
                wait_gather(j)

                @pl.when(p >= 3)
                def _():
                    wait_out(j)

                compute(j)
                fire_out(p, j)
        return carry

    lax.fori_loop(0, 22, step_body, 0)
    wait_out(0)


_sc_call = functools.partial(
    pl.kernel,
    out_type=jax.ShapeDtypeStruct((BATCH, T, OUT_D), jnp.float32),
    mesh=plsc.VectorSubcoreMesh(core_axis_name="c", subcore_axis_name="s"),
    compiler_params=pltpu.CompilerParams(use_tc_tiling_on_sc=False),
    scratch_types=(
        [pltpu.VMEM((1, IDX_PER_PAIR), jnp.int32)] * 3
        + [pltpu.VMEM((IDX_PER_PAIR, EMBED_DIM), jnp.float32)] * 3
        + [pltpu.VMEM((2, T, OUT_D), jnp.float32)] * 3
        + [pltpu.SemaphoreType.DMA] * 9
    ),
)(_body)


def kernel(x, table):
    x2 = x.reshape(BATCH * T * NF // IDX_PER_PAIR, IDX_PER_PAIR)
    return _sc_call(x2, table)
